# Initial kernel scaffold; baseline (speedup 1.0000x reference)
import functools

import jax
import jax.numpy as jnp
from jax import lax
from jax.experimental import pallas as pl
from jax.experimental.pallas import tpu as pltpu
from jax.experimental.pallas import tpu_sc as plsc

_N = 10000
_D = 128
_E = 320000
_NC = 2
_NS = 16
_NW = _NC * _NS
_EPW = _E // _NW
_CH = 80
_NCHUNK = _EPW // _CH
_RPT = _N // _NS
_ZR = 125
_LG = _D // 16


def _spmm_body(x_hbm, cols_hbm, rows_hbm, vals_hbm, out_hbm,
               cols_v, rows_v, vals_v, gbuf, stage, acc_sh, sem):
    c = lax.axis_index("c")
    s = lax.axis_index("s")
    wid = c * _NS + s
    base = wid * _EPW

    def _ld(j, _):
        pltpu.sync_copy(cols_hbm.at[pl.ds(base + j * _CH, _CH)], cols_v.at[j])
        pltpu.sync_copy(rows_hbm.at[pl.ds(base + j * _CH, _CH)], rows_v.at[j])
        pltpu.sync_copy(vals_hbm.at[pl.ds(base + j * _CH, _CH)], vals_v.at[j])
        return 0
    lax.fori_loop(0, _NCHUNK, _ld, 0)

    zero = jnp.zeros((16,), jnp.float32)

    def _zb(r, _):
        for g in range(_LG):
            stage[r, pl.ds(g * 16, 16)] = zero
        return 0
    lax.fori_loop(0, _ZR, _zb, 0)
    for k in range(_RPT // _ZR):
        pltpu.sync_copy(stage, acc_sh.at[pl.ds(s * _RPT + k * _ZR, _ZR)])
    plsc.subcore_barrier()

    def _chunk(j, _):
        pltpu.async_copy(x_hbm.at[cols_v.at[j]], gbuf, sem).wait()

        def _edge(e, _):
            v = vals_v[j, e]
            for g in range(_LG):
                gbuf[e, pl.ds(g * 16, 16)] = gbuf[e, pl.ds(g * 16, 16)] * v
            return 0
        lax.fori_loop(0, _CH, _edge, 0)
        pltpu.sync_copy(gbuf, acc_sh.at[rows_v.at[j]], add=True)
        return 0
    lax.fori_loop(0, _NCHUNK, _chunk, 0)
    plsc.subcore_barrier()

    for k in range(_RPT // _ZR):
        sl = pl.ds(s * _RPT + k * _ZR, _ZR)
        pltpu.sync_copy(acc_sh.at[sl], stage)
        pltpu.sync_copy(stage,
                        out_hbm.at[pl.ds(c * _N + s * _RPT + k * _ZR, _ZR)])


_spmm = functools.partial(
    pl.kernel,
    out_type=jax.ShapeDtypeStruct((_NC * _N, _D), jnp.float32),
    mesh=plsc.VectorSubcoreMesh(
        core_axis_name="c", subcore_axis_name="s",
        num_cores=_NC, num_subcores=_NS),
    scratch_types=[
        pltpu.VMEM((_NCHUNK, _CH), jnp.int32),
        pltpu.VMEM((_NCHUNK, _CH), jnp.int32),
        pltpu.VMEM((_NCHUNK, _CH), jnp.float32),
        pltpu.VMEM((_CH, _D), jnp.float32),
        pltpu.VMEM((_ZR, _D), jnp.float32),
        pltpu.VMEM_SHARED((_N, _D), jnp.float32),
        pltpu.SemaphoreType.DMA,
    ],
)(_spmm_body)


_BLK = 2000


def _add2_body(a, b, o):
    o[...] = a[...] + b[...]


def _add3_body(a, b, r, o):
    o[...] = a[...] + b[...] + r[...]


def _final_body(a, b, x1, x0, o):
    x2 = a[...] + b[...] + x1[...]
    o[...] = (x0[...] + x1[...] + x2) / 3.0


def _ew_call(body, n_in):
    spec = pl.BlockSpec((_BLK, _D), lambda i: (i, 0))
    return pl.pallas_call(
        body,
        out_shape=jax.ShapeDtypeStruct((_N, _D), jnp.float32),
        grid=(_N // _BLK,),
        in_specs=[spec] * n_in,
        out_specs=spec,
    )


_add2 = _ew_call(_add2_body, 2)
_add3 = _ew_call(_add3_body, 3)
_final = _ew_call(_final_body, 4)


@jax.jit
def _run(x0, src_rows, src_cols, src_vals, tar_rows, tar_cols, tar_vals):
    pt = _spmm(x0, tar_cols, tar_rows, tar_vals)
    mt = _add2(pt[:_N], pt[_N:])
    ps = _spmm(mt, src_cols, src_rows, src_vals)
    x1 = _add3(ps[:_N], ps[_N:], x0)
    qt = _spmm(x1, tar_cols, tar_rows, tar_vals)
    mt2 = _add2(qt[:_N], qt[_N:])
    qs = _spmm(mt2, src_cols, src_rows, src_vals)
    return _final(qs[:_N], qs[_N:], x1, x0)


def kernel(pois_embs, HG_src_indices, HG_src_values, HG_tar_indices,
           HG_tar_values):
    src = HG_src_indices.astype(jnp.int32)
    tar = HG_tar_indices.astype(jnp.int32)
    return _run(pois_embs, src[0], src[1], HG_src_values,
                tar[0], tar[1], HG_tar_values)


# speedup vs baseline: 3.3891x; 3.3891x over previous
import functools

import jax
import jax.numpy as jnp
from jax import lax
from jax.experimental import pallas as pl
from jax.experimental.pallas import tpu as pltpu
from jax.experimental.pallas import tpu_sc as plsc

_N = 10000
_D = 128
_E = 320000
_NC = 2
_NS = 16
_NW = _NC * _NS
_EPW = _E // _NW
_CH = 128
_NFULL = _EPW // _CH
_TAIL = _EPW - _NFULL * _CH
_NCHUNK = _NFULL + 1
_RCH = 80
_NRCH = _N // _RCH
_KMAX = -(-_NRCH // _NS)
_LG = _D // 16


def _spmm_body(x_hbm, cols_hbm, rows_hbm, vals_hbm, out_hbm,
               cols_v, rows_v, vals_v, gbuf, acc_sh, sem):
    c = lax.axis_index("c")
    s = lax.axis_index("s")
    wid = c * _NS + s
    base = wid * _EPW

    def _ld(j, _):
        pltpu.sync_copy(cols_hbm.at[pl.ds(base + j * _CH, _CH)], cols_v.at[j])
        pltpu.sync_copy(rows_hbm.at[pl.ds(base + j * _CH, _CH)], rows_v.at[j])
        pltpu.sync_copy(vals_hbm.at[pl.ds(base + j * _CH, _CH)], vals_v.at[j])
        return 0
    lax.fori_loop(0, _NFULL, _ld, 0)

    tb = pl.ds(base + _NFULL * _CH, _TAIL)
    pltpu.sync_copy(cols_hbm.at[tb], cols_v.at[_NFULL, pl.ds(0, _TAIL)])
    pltpu.sync_copy(rows_hbm.at[tb], rows_v.at[_NFULL, pl.ds(0, _TAIL)])
    pltpu.sync_copy(vals_hbm.at[tb], vals_v.at[_NFULL, pl.ds(0, _TAIL)])
    izero = jnp.zeros((16,), jnp.int32)
    fzero = jnp.zeros((16,), jnp.float32)
    for t in range(_TAIL, _CH, 16):
        cols_v[_NFULL, pl.ds(t, 16)] = izero
        rows_v[_NFULL, pl.ds(t, 16)] = izero
        vals_v[_NFULL, pl.ds(t, 16)] = fzero

    def _zb(r, _):
        for g in range(_LG):
            gbuf[r, pl.ds(g * 16, 16)] = fzero
        return 0
    lax.fori_loop(0, _RCH, _zb, 0)
    for k in range(_KMAX):
        cid = s + k * _NS

        @pl.when(cid < _NRCH)
        def _():
            pltpu.sync_copy(gbuf.at[pl.ds(0, _RCH)],
                            acc_sh.at[pl.ds(cid * _RCH, _RCH)])
    plsc.subcore_barrier()

    def _chunk(j, _):
        pltpu.async_copy(x_hbm.at[cols_v.at[j]], gbuf, sem).wait()

        def _edge16(eb, _):
            vvec = vals_v[j, pl.ds(eb * 16, 16)]
            for l in range(16):
                e = eb * 16 + l
                v = vvec[l]
                for g in range(_LG):
                    gbuf[e, pl.ds(g * 16, 16)] = gbuf[e, pl.ds(g * 16, 16)] * v
            return 0
        lax.fori_loop(0, _CH // 16, _edge16, 0)
        pltpu.sync_copy(gbuf, acc_sh.at[rows_v.at[j]], add=True)
        return 0
    lax.fori_loop(0, _NCHUNK, _chunk, 0)
    plsc.subcore_barrier()

    for k in range(_KMAX):
        cid = s + k * _NS

        @pl.when(cid < _NRCH)
        def _():
            pltpu.sync_copy(acc_sh.at[pl.ds(cid * _RCH, _RCH)],
                            gbuf.at[pl.ds(0, _RCH)])
            pltpu.sync_copy(gbuf.at[pl.ds(0, _RCH)],
                            out_hbm.at[pl.ds(c * _N + cid * _RCH, _RCH)])


_spmm = functools.partial(
    pl.kernel,
    out_type=jax.ShapeDtypeStruct((_NC * _N, _D), jnp.float32),
    mesh=plsc.VectorSubcoreMesh(
        core_axis_name="c", subcore_axis_name="s",
        num_cores=_NC, num_subcores=_NS),
    scratch_types=[
        pltpu.VMEM((_NCHUNK, _CH), jnp.int32),
        pltpu.VMEM((_NCHUNK, _CH), jnp.int32),
        pltpu.VMEM((_NCHUNK, _CH), jnp.float32),
        pltpu.VMEM((_CH, _D), jnp.float32),
        pltpu.VMEM_SHARED((_N, _D), jnp.float32),
        pltpu.SemaphoreType.DMA,
    ],
)(_spmm_body)


_BLK = 2000


def _add2_body(a, b, o):
    o[...] = a[...] + b[...]


def _add3_body(a, b, r, o):
    o[...] = a[...] + b[...] + r[...]


def _final_body(a, b, x1, x0, o):
    x2 = a[...] + b[...] + x1[...]
    o[...] = (x0[...] + x1[...] + x2) / 3.0


def _ew_call(body, n_in):
    spec = pl.BlockSpec((_BLK, _D), lambda i: (i, 0))
    return pl.pallas_call(
        body,
        out_shape=jax.ShapeDtypeStruct((_N, _D), jnp.float32),
        grid=(_N // _BLK,),
        in_specs=[spec] * n_in,
        out_specs=spec,
    )


_add2 = _ew_call(_add2_body, 2)
_add3 = _ew_call(_add3_body, 3)
_final = _ew_call(_final_body, 4)


@jax.jit
def _run(x0, src_rows, src_cols, src_vals, tar_rows, tar_cols, tar_vals):
    pt = _spmm(x0, tar_cols, tar_rows, tar_vals)
    mt = _add2(pt[:_N], pt[_N:])
    ps = _spmm(mt, src_cols, src_rows, src_vals)
    x1 = _add3(ps[:_N], ps[_N:], x0)
    qt = _spmm(x1, tar_cols, tar_rows, tar_vals)
    mt2 = _add2(qt[:_N], qt[_N:])
    qs = _spmm(mt2, src_cols, src_rows, src_vals)
    return _final(qs[:_N], qs[_N:], x1, x0)


def kernel(pois_embs, HG_src_indices, HG_src_values, HG_tar_indices,
           HG_tar_values):
    src = HG_src_indices.astype(jnp.int32)
    tar = HG_tar_indices.astype(jnp.int32)
    return _run(pois_embs, src[0], src[1], HG_src_values,
                tar[0], tar[1], HG_tar_values)


# speedup vs baseline: 5.1461x; 1.5185x over previous
import functools

import jax
import jax.numpy as jnp
from jax import lax
from jax.experimental import pallas as pl
from jax.experimental.pallas import tpu as pltpu
from jax.experimental.pallas import tpu_sc as plsc

_N = 10000
_D = 128
_E = 320000
_NC = 2
_NS = 16
_NW = _NC * _NS
_EPW = _E // _NW
_CH = 128
_NFULL = _EPW // _CH
_TAIL = _EPW - _NFULL * _CH
_RCH = 80
_NRCH = _N // _RCH
_KMAX = -(-_NRCH // _NS)
_LG = _D // 16


def _spmm_body(x_hbm, cols_hbm, rows_hbm, vals_hbm, out_hbm,
               cols_r, rows_r, vals_r, gbuf0, gbuf1, acc_sh,
               gsem, csem, rsem, vsem):
    c = lax.axis_index("c")
    s = lax.axis_index("s")
    wid = c * _NS + s
    base = wid * _EPW
    gbufs = (gbuf0, gbuf1)

    def _fire_idx(j, slot):
        sl = pl.ds(base + j * _CH, _CH)
        pltpu.async_copy(cols_hbm.at[sl], cols_r.at[slot], csem.at[slot])
        pltpu.async_copy(rows_hbm.at[sl], rows_r.at[slot], rsem.at[slot])
        pltpu.async_copy(vals_hbm.at[sl], vals_r.at[slot], vsem.at[slot])

    def _drain_idx(j, slot):
        sl = pl.ds(base + j * _CH, _CH)
        pltpu.make_async_copy(cols_hbm.at[sl], cols_r.at[slot],
                              csem.at[slot]).wait()
        pltpu.make_async_copy(rows_hbm.at[sl], rows_r.at[slot],
                              rsem.at[slot]).wait()
        pltpu.make_async_copy(vals_hbm.at[sl], vals_r.at[slot],
                              vsem.at[slot]).wait()

    def _scale(gb, p):
        def _edge16(eb, _):
            vvec = vals_r[p, pl.ds(eb * 16, 16)]
            for l in range(16):
                e = eb * 16 + l
                v = vvec[l]
                for g in range(_LG):
                    gb[e, pl.ds(g * 16, 16)] = gb[e, pl.ds(g * 16, 16)] * v
            return 0
        lax.fori_loop(0, _CH // 16, _edge16, 0)

    fzero = jnp.zeros((16,), jnp.float32)

    def _zb(r, _):
        for g in range(_LG):
            gbuf0[r, pl.ds(g * 16, 16)] = fzero
        return 0
    lax.fori_loop(0, _RCH, _zb, 0)
    for k in range(_KMAX):
        cid = s + k * _NS

        @pl.when(cid < _NRCH)
        def _():
            pltpu.sync_copy(gbuf0.at[pl.ds(0, _RCH)],
                            acc_sh.at[pl.ds(cid * _RCH, _RCH)])
    plsc.subcore_barrier()

    _fire_idx(0, 0)
    _drain_idx(0, 0)
    pltpu.async_copy(x_hbm.at[cols_r.at[0]], gbuf0, gsem.at[0])
    _fire_idx(1, 1)

    def _pair(jj, _):
        for p in range(2):
            j = jj * 2 + p
            gb = gbufs[p]
            pltpu.make_async_copy(x_hbm.at[cols_r.at[p]], gb,
                                  gsem.at[p]).wait()

            @pl.when(j + 1 < _NFULL)
            def _():
                _drain_idx(j + 1, 1 - p)
                pltpu.async_copy(x_hbm.at[cols_r.at[1 - p]], gbufs[1 - p],
                                 gsem.at[1 - p])
            _scale(gb, p)
            pltpu.sync_copy(gb, acc_sh.at[rows_r.at[p]], add=True)

            @pl.when(j + 2 < _NFULL)
            def _():
                _fire_idx(j + 2, p)
        return 0
    lax.fori_loop(0, _NFULL // 2, _pair, 0)

    tb = pl.ds(base + _NFULL * _CH, _TAIL)
    pltpu.sync_copy(cols_hbm.at[tb], cols_r.at[0, pl.ds(0, _TAIL)])
    pltpu.sync_copy(rows_hbm.at[tb], rows_r.at[0, pl.ds(0, _TAIL)])
    pltpu.sync_copy(vals_hbm.at[tb], vals_r.at[0, pl.ds(0, _TAIL)])
    izero = jnp.zeros((16,), jnp.int32)
    for t in range(_TAIL, _CH, 16):
        cols_r[0, pl.ds(t, 16)] = izero
        rows_r[0, pl.ds(t, 16)] = izero
        vals_r[0, pl.ds(t, 16)] = fzero
    pltpu.async_copy(x_hbm.at[cols_r.at[0]], gbuf0, gsem.at[0]).wait()
    _scale(gbuf0, 0)
    pltpu.sync_copy(gbuf0, acc_sh.at[rows_r.at[0]], add=True)
    plsc.subcore_barrier()

    for k in range(_KMAX):
        cid = s + k * _NS

        @pl.when(cid < _NRCH)
        def _():
            pltpu.sync_copy(acc_sh.at[pl.ds(cid * _RCH, _RCH)],
                            gbuf0.at[pl.ds(0, _RCH)])
            pltpu.sync_copy(gbuf0.at[pl.ds(0, _RCH)],
                            out_hbm.at[pl.ds(c * _N + cid * _RCH, _RCH)])


_spmm = functools.partial(
    pl.kernel,
    out_type=jax.ShapeDtypeStruct((_NC * _N, _D), jnp.float32),
    mesh=plsc.VectorSubcoreMesh(
        core_axis_name="c", subcore_axis_name="s",
        num_cores=_NC, num_subcores=_NS),
    scratch_types=[
        pltpu.VMEM((2, _CH), jnp.int32),
        pltpu.VMEM((2, _CH), jnp.int32),
        pltpu.VMEM((2, _CH), jnp.float32),
        pltpu.VMEM((_CH, _D), jnp.float32),
        pltpu.VMEM((_CH, _D), jnp.float32),
        pltpu.VMEM_SHARED((_N, _D), jnp.float32),
        pltpu.SemaphoreType.DMA((2,)),
        pltpu.SemaphoreType.DMA((2,)),
        pltpu.SemaphoreType.DMA((2,)),
        pltpu.SemaphoreType.DMA((2,)),
    ],
)(_spmm_body)


_BLK = 2000


def _add2_body(a, b, o):
    o[...] = a[...] + b[...]


def _add3_body(a, b, r, o):
    o[...] = a[...] + b[...] + r[...]


def _final_body(a, b, x1, x0, o):
    x2 = a[...] + b[...] + x1[...]
    o[...] = (x0[...] + x1[...] + x2) / 3.0


def _ew_call(body, n_in):
    spec = pl.BlockSpec((_BLK, _D), lambda i: (i, 0))
    return pl.pallas_call(
        body,
        out_shape=jax.ShapeDtypeStruct((_N, _D), jnp.float32),
        grid=(_N // _BLK,),
        in_specs=[spec] * n_in,
        out_specs=spec,
    )


_add2 = _ew_call(_add2_body, 2)
_add3 = _ew_call(_add3_body, 3)
_final = _ew_call(_final_body, 4)


@jax.jit
def _run(x0, src_rows, src_cols, src_vals, tar_rows, tar_cols, tar_vals):
    pt = _spmm(x0, tar_cols, tar_rows, tar_vals)
    mt = _add2(pt[:_N], pt[_N:])
    ps = _spmm(mt, src_cols, src_rows, src_vals)
    x1 = _add3(ps[:_N], ps[_N:], x0)
    qt = _spmm(x1, tar_cols, tar_rows, tar_vals)
    mt2 = _add2(qt[:_N], qt[_N:])
    qs = _spmm(mt2, src_cols, src_rows, src_vals)
    return _final(qs[:_N], qs[_N:], x1, x0)


def kernel(pois_embs, HG_src_indices, HG_src_values, HG_tar_indices,
           HG_tar_values):
    src = HG_src_indices.astype(jnp.int32)
    tar = HG_tar_indices.astype(jnp.int32)
    return _run(pois_embs, src[0], src[1], HG_src_values,
                tar[0], tar[1], HG_tar_values)


# speedup vs baseline: 5.6377x; 1.0955x over previous
import functools

import jax
import jax.numpy as jnp
from jax import lax
from jax.experimental import pallas as pl
from jax.experimental.pallas import tpu as pltpu
from jax.experimental.pallas import tpu_sc as plsc

_N = 10000
_D = 128
_E = 320000
_NC = 2
_NS = 16
_NW = _NC * _NS
_EPW = _E // _NW
_CH = 128
_NFULL = _EPW // _CH
_TAIL = _EPW - _NFULL * _CH
_RCH = 80
_NRCH = _N // _RCH
_KMAX = -(-_NRCH // _NS)
_LG = _D // 16


def _spmm_body(x_hbm, cols_hbm, rows_hbm, vals_hbm, out_hbm,
               cols_r, rows_r, vals_r, rif, gbuf0, gbuf1, acc_sh,
               gsem, csem, rsem, vsem, ssem):
    c = lax.axis_index("c")
    s = lax.axis_index("s")
    wid = c * _NS + s
    base = wid * _EPW
    gbufs = (gbuf0, gbuf1)

    def _fire_idx(j, slot):
        sl = pl.ds(base + j * _CH, _CH)
        pltpu.async_copy(cols_hbm.at[sl], cols_r.at[slot], csem.at[slot])
        pltpu.async_copy(rows_hbm.at[sl], rows_r.at[slot], rsem.at[slot])
        pltpu.async_copy(vals_hbm.at[sl], vals_r.at[slot], vsem.at[slot])

    def _drain_idx(j, slot):
        sl = pl.ds(base + j * _CH, _CH)
        pltpu.make_async_copy(cols_hbm.at[sl], cols_r.at[slot],
                              csem.at[slot]).wait()
        pltpu.make_async_copy(rows_hbm.at[sl], rows_r.at[slot],
                              rsem.at[slot]).wait()
        pltpu.make_async_copy(vals_hbm.at[sl], vals_r.at[slot],
                              vsem.at[slot]).wait()

    def _scale(gb, p):
        def _edge16(eb, _):
            vvec = vals_r[p, pl.ds(eb * 16, 16)]
            for l in range(16):
                e = eb * 16 + l
                v = vvec[l]
                for g in range(_LG):
                    gb[e, pl.ds(g * 16, 16)] = gb[e, pl.ds(g * 16, 16)] * v
            return 0
        lax.fori_loop(0, _CH // 16, _edge16, 0)

    fzero = jnp.zeros((16,), jnp.float32)

    def _zb(r, _):
        for g in range(_LG):
            gbuf0[r, pl.ds(g * 16, 16)] = fzero
        return 0
    lax.fori_loop(0, _RCH, _zb, 0)
    for k in range(_KMAX):
        cid = s + k * _NS

        @pl.when(cid < _NRCH)
        def _():
            pltpu.sync_copy(gbuf0.at[pl.ds(0, _RCH)],
                            acc_sh.at[pl.ds(cid * _RCH, _RCH)])
    plsc.subcore_barrier()

    _fire_idx(0, 0)
    _drain_idx(0, 0)
    pltpu.async_copy(x_hbm.at[cols_r.at[0]], gbuf0, gsem.at[0])
    _fire_idx(1, 1)

    def _pair(jj, _):
        for p in range(2):
            j = jj * 2 + p
            gb = gbufs[p]
            pltpu.make_async_copy(x_hbm.at[cols_r.at[p]], gb,
                                  gsem.at[p]).wait()

            @pl.when(j >= 1)
            def _():
                pltpu.make_async_copy(
                    gbufs[1 - p], acc_sh.at[rif.at[1 - p]],
                    ssem.at[1 - p]).wait()

            @pl.when(j + 1 < _NFULL)
            def _():
                _drain_idx(j + 1, 1 - p)
                pltpu.async_copy(x_hbm.at[cols_r.at[1 - p]], gbufs[1 - p],
                                 gsem.at[1 - p])
            _scale(gb, p)
            for g in range(_LG):
                rif[p, pl.ds(g * 16, 16)] = rows_r[p, pl.ds(g * 16, 16)]
            pltpu.async_copy(gb, acc_sh.at[rif.at[p]], ssem.at[p], add=True)

            @pl.when(j + 2 < _NFULL)
            def _():
                _fire_idx(j + 2, p)
        return 0
    lax.fori_loop(0, _NFULL // 2, _pair, 0)
    pltpu.make_async_copy(gbuf1, acc_sh.at[rif.at[1]], ssem.at[1]).wait()

    tb = pl.ds(base + _NFULL * _CH, _TAIL)
    pltpu.sync_copy(cols_hbm.at[tb], cols_r.at[0, pl.ds(0, _TAIL)])
    pltpu.sync_copy(rows_hbm.at[tb], rows_r.at[0, pl.ds(0, _TAIL)])
    pltpu.sync_copy(vals_hbm.at[tb], vals_r.at[0, pl.ds(0, _TAIL)])
    izero = jnp.zeros((16,), jnp.int32)
    for t in range(_TAIL, _CH, 16):
        cols_r[0, pl.ds(t, 16)] = izero
        rows_r[0, pl.ds(t, 16)] = izero
        vals_r[0, pl.ds(t, 16)] = fzero
    pltpu.async_copy(x_hbm.at[cols_r.at[0]], gbuf0, gsem.at[0]).wait()
    _scale(gbuf0, 0)
    pltpu.sync_copy(gbuf0, acc_sh.at[rows_r.at[0]], add=True)
    plsc.subcore_barrier()

    for k in range(_KMAX):
        cid = s + k * _NS

        @pl.when(cid < _NRCH)
        def _():
            pltpu.sync_copy(acc_sh.at[pl.ds(cid * _RCH, _RCH)],
                            gbuf0.at[pl.ds(0, _RCH)])
            pltpu.sync_copy(gbuf0.at[pl.ds(0, _RCH)],
                            out_hbm.at[pl.ds(c * _N + cid * _RCH, _RCH)])


_spmm = functools.partial(
    pl.kernel,
    out_type=jax.ShapeDtypeStruct((_NC * _N, _D), jnp.float32),
    mesh=plsc.VectorSubcoreMesh(
        core_axis_name="c", subcore_axis_name="s",
        num_cores=_NC, num_subcores=_NS),
    scratch_types=[
        pltpu.VMEM((2, _CH), jnp.int32),
        pltpu.VMEM((2, _CH), jnp.int32),
        pltpu.VMEM((2, _CH), jnp.float32),
        pltpu.VMEM((2, _CH), jnp.int32),
        pltpu.VMEM((_CH, _D), jnp.float32),
        pltpu.VMEM((_CH, _D), jnp.float32),
        pltpu.VMEM_SHARED((_N, _D), jnp.float32),
        pltpu.SemaphoreType.DMA((2,)),
        pltpu.SemaphoreType.DMA((2,)),
        pltpu.SemaphoreType.DMA((2,)),
        pltpu.SemaphoreType.DMA((2,)),
        pltpu.SemaphoreType.DMA((2,)),
    ],
)(_spmm_body)


_BLK = 2000


def _add2_body(a, b, o):
    o[...] = a[...] + b[...]


def _add3_body(a, b, r, o):
    o[...] = a[...] + b[...] + r[...]


def _final_body(a, b, x1, x0, o):
    x2 = a[...] + b[...] + x1[...]
    o[...] = (x0[...] + x1[...] + x2) / 3.0


def _ew_call(body, n_in):
    spec = pl.BlockSpec((_BLK, _D), lambda i: (i, 0))
    return pl.pallas_call(
        body,
        out_shape=jax.ShapeDtypeStruct((_N, _D), jnp.float32),
        grid=(_N // _BLK,),
        in_specs=[spec] * n_in,
        out_specs=spec,
    )


_add2 = _ew_call(_add2_body, 2)
_add3 = _ew_call(_add3_body, 3)
_final = _ew_call(_final_body, 4)


@jax.jit
def _run(x0, src_rows, src_cols, src_vals, tar_rows, tar_cols, tar_vals):
    pt = _spmm(x0, tar_cols, tar_rows, tar_vals)
    mt = _add2(pt[:_N], pt[_N:])
    ps = _spmm(mt, src_cols, src_rows, src_vals)
    x1 = _add3(ps[:_N], ps[_N:], x0)
    qt = _spmm(x1, tar_cols, tar_rows, tar_vals)
    mt2 = _add2(qt[:_N], qt[_N:])
    qs = _spmm(mt2, src_cols, src_rows, src_vals)
    return _final(qs[:_N], qs[_N:], x1, x0)


def kernel(pois_embs, HG_src_indices, HG_src_values, HG_tar_indices,
           HG_tar_values):
    src = HG_src_indices.astype(jnp.int32)
    tar = HG_tar_indices.astype(jnp.int32)
    return _run(pois_embs, src[0], src[1], HG_src_values,
                tar[0], tar[1], HG_tar_values)
